# baseline (speedup 1.0000x reference)
import functools

import jax
import jax.numpy as jnp
from jax import lax
from jax.experimental import pallas as pl
from jax.experimental.pallas import tpu as pltpu
from jax.experimental.pallas import tpu_sc as plsc

_NC = 2
_NS = 16
_CHUNK = 80
_GRP = 2


def _sc_segsum(xf, src_flat, dst_flat, zeros_rows, n_nodes, feat):
    num_edges = src_flat.shape[0]
    e_tile = num_edges // (_NC * _NS)
    n_chunks = e_tile // _CHUNK
    n_groups = n_chunks // _GRP
    n_tail = n_chunks - n_groups * _GRP
    mesh = plsc.VectorSubcoreMesh(
        core_axis_name="c", subcore_axis_name="s",
        num_cores=_NC, num_subcores=_NS)

    @functools.partial(
        pl.kernel,
        out_type=jax.ShapeDtypeStruct((_NC, n_nodes, feat), jnp.float32),
        mesh=mesh,
        scratch_types=[
            pltpu.VMEM_SHARED((n_nodes, feat), jnp.float32),
            pltpu.VMEM((4 * _GRP, _CHUNK), jnp.int32),
            pltpu.VMEM((4 * _GRP, _CHUNK), jnp.int32),
            pltpu.VMEM((2 * _GRP, _CHUNK, feat), jnp.float32),
            pltpu.SemaphoreType.DMA, pltpu.SemaphoreType.DMA,
            pltpu.SemaphoreType.DMA, pltpu.SemaphoreType.DMA,
            pltpu.SemaphoreType.DMA, pltpu.SemaphoreType.DMA,
        ],
    )
    def k(x_hbm, src_hbm, dst_hbm, z_hbm, out_hbm, agg,
          src_v, dst_v, bufs, isem0, isem1, gsem0, gsem1, ssem0, ssem1):
        c = lax.axis_index("c")
        s = lax.axis_index("s")
        base = (c * _NS + s) * e_tile

        @pl.when(s < n_nodes // 1000)
        def _zero():
            zbase = pl.multiple_of(s * 1000, 8)
            pltpu.sync_copy(z_hbm, agg.at[pl.ds(zbase, 1000)])

        plsc.subcore_barrier()

        def fetch_idx(g, isem):
            r = (g % 4) * _GRP
            for b in range(_GRP):
                off = base + (g * _GRP + b) * _CHUNK
                pltpu.async_copy(src_hbm.at[pl.ds(off, _CHUNK)],
                                 src_v.at[r + b], isem)
                pltpu.async_copy(dst_hbm.at[pl.ds(off, _CHUNK)],
                                 dst_v.at[r + b], isem)

        def wait_idx(g, isem):
            r = (g % 4) * _GRP
            for b in range(_GRP):
                pltpu.make_async_copy(src_hbm.at[pl.ds(base, _CHUNK)],
                                      src_v.at[r + b], isem).wait()
                pltpu.make_async_copy(dst_hbm.at[pl.ds(base, _CHUNK)],
                                      dst_v.at[r + b], isem).wait()

        def fire_rows(g, gsem):
            ri = (g % 4) * _GRP
            rb = (g % 2) * _GRP
            for b in range(_GRP):
                pltpu.async_copy(x_hbm.at[src_v.at[ri + b]],
                                 bufs.at[rb + b], gsem)

        def wait_rows(g, gsem):
            ri = (g % 4) * _GRP
            rb = (g % 2) * _GRP
            for b in range(_GRP):
                pltpu.make_async_copy(x_hbm.at[src_v.at[ri + b]],
                                      bufs.at[rb + b], gsem).wait()

        def fire_scatters(g, ssem):
            ri = (g % 4) * _GRP
            rb = (g % 2) * _GRP
            for b in range(_GRP):
                pltpu.async_copy(bufs.at[rb + b],
                                 agg.at[dst_v.at[ri + b]], ssem, add=True)

        def wait_scatters(g, ssem):
            ri = (g % 4) * _GRP
            rb = (g % 2) * _GRP
            for b in range(_GRP):
                pltpu.make_async_copy(bufs.at[rb + b],
                                      agg.at[dst_v.at[ri + b]], ssem).wait()

        fetch_idx(0, isem0)
        fetch_idx(1, isem1)
        wait_idx(0, isem0)
        fire_rows(0, gsem0)

        def group_body(g, carry):
            def run(gsem, ssem, gsem_o, ssem_o, isem_n, isem_n2):
                wait_rows(g, gsem)
                fire_scatters(g, ssem)

                @pl.when(g >= 1)
                def _free_other():
                    wait_scatters(g - 1, ssem_o)

                @pl.when(g + 1 < n_groups)
                def _next_rows():
                    wait_idx(g + 1, isem_n)
                    fire_rows(g + 1, gsem_o)

                @pl.when(g + 2 < n_groups)
                def _next_idx():
                    fetch_idx(g + 2, isem_n2)

            @pl.when(g % 2 == 0)
            def _even():
                run(gsem0, ssem0, gsem1, ssem1, isem1, isem0)

            @pl.when(g % 2 == 1)
            def _odd():
                run(gsem1, ssem1, gsem0, ssem0, isem0, isem1)

            return carry

        lax.fori_loop(0, n_groups, group_body, 0)
        last = n_groups - 1
        wait_scatters(last, ssem1 if last % 2 == 1 else ssem0)
        for t in range(n_tail):
            off = base + (n_groups * _GRP + t) * _CHUNK
            pltpu.sync_copy(src_hbm.at[pl.ds(off, _CHUNK)], src_v.at[0])
            pltpu.sync_copy(dst_hbm.at[pl.ds(off, _CHUNK)], dst_v.at[0])
            pltpu.sync_copy(x_hbm.at[src_v.at[0]], bufs.at[0])
            pltpu.sync_copy(bufs.at[0], agg.at[dst_v.at[0]], add=True)
        plsc.subcore_barrier()

        @pl.when(s < n_nodes // 1000)
        def _writeback():
            wbase = pl.multiple_of(s * 1000, 8)
            pltpu.sync_copy(agg.at[pl.ds(wbase, 1000)],
                            out_hbm.at[c, pl.ds(wbase, 1000)])

    return k(xf, src_flat, dst_flat, zeros_rows)


def _tc_root(h, w_root_t, bias, blk=1000):
    n, feat = h.shape

    def body(h_ref, w_ref, b_ref, o_ref):
        o_ref[...] = jnp.dot(h_ref[...], w_ref[...],
                             preferred_element_type=jnp.float32) + b_ref[...]

    return pl.pallas_call(
        body,
        grid=(n // blk,),
        in_specs=[
            pl.BlockSpec((blk, feat), lambda i: (i, 0)),
            pl.BlockSpec((feat, feat), lambda i: (0, 0)),
            pl.BlockSpec((1, feat), lambda i: (0, 0)),
        ],
        out_specs=pl.BlockSpec((blk, feat), lambda i: (i, 0)),
        out_shape=jax.ShapeDtypeStruct((n, feat), jnp.float32),
    )(h, w_root_t, bias.reshape(1, feat))


def _tc_combine_root(agg2, r, w_rel_t, w_root_next_t, b_next, blk=1000):
    n, feat = r.shape

    def body(a0_ref, a1_ref, r_ref, w_ref, wn_ref, bn_ref, h_ref, rn_ref):
        acc = jnp.dot(a0_ref[0] + a1_ref[0], w_ref[...],
                      preferred_element_type=jnp.float32)
        h_blk = jnp.maximum(acc + r_ref[...], 0.0)
        h_ref[...] = h_blk
        rn_ref[...] = jnp.dot(h_blk, wn_ref[...],
                              preferred_element_type=jnp.float32) + bn_ref[...]

    return pl.pallas_call(
        body,
        grid=(n // blk,),
        in_specs=[
            pl.BlockSpec((1, blk, feat), lambda i: (0, i, 0)),
            pl.BlockSpec((1, blk, feat), lambda i: (1, i, 0)),
            pl.BlockSpec((blk, feat), lambda i: (i, 0)),
            pl.BlockSpec((feat, feat), lambda i: (0, 0)),
            pl.BlockSpec((feat, feat), lambda i: (0, 0)),
            pl.BlockSpec((1, feat), lambda i: (0, 0)),
        ],
        out_specs=[
            pl.BlockSpec((blk, feat), lambda i: (i, 0)),
            pl.BlockSpec((blk, feat), lambda i: (i, 0)),
        ],
        out_shape=[
            jax.ShapeDtypeStruct((n, feat), jnp.float32),
            jax.ShapeDtypeStruct((n, feat), jnp.float32),
        ],
    )(agg2, agg2, r, w_rel_t, w_root_next_t, b_next.reshape(1, feat))


def _tc_combine_pool_linear(agg2, r, w_rel_t, batch_r, w_lin_t_pad, b_lin_pad,
                            num_graphs, blk=1000):
    n, feat = r.shape
    nblk = n // blk

    def body(a0_ref, a1_ref, r_ref, w_ref, bid_ref, wl_ref, bl_ref,
             o_ref, sums, counts):
        i = pl.program_id(0)

        @pl.when(i == 0)
        def _init():
            sums[...] = jnp.zeros_like(sums)
            counts[...] = jnp.zeros_like(counts)

        h_blk = jnp.dot(a0_ref[0] + a1_ref[0], w_ref[...],
                        preferred_element_type=jnp.float32) + r_ref[...]

        bid = bid_ref[0]
        gids = lax.broadcasted_iota(jnp.int32, (num_graphs, blk), 0)
        onehot_t = (gids == bid).astype(jnp.float32)
        sums[...] += lax.dot_general(
            onehot_t, h_blk, (((1,), (0,)), ((), ())),
            preferred_element_type=jnp.float32)
        counts[...] += lax.dot_general(
            onehot_t, jnp.ones((blk, 8), jnp.float32),
            (((1,), (0,)), ((), ())), preferred_element_type=jnp.float32)

        @pl.when(i == nblk - 1)
        def _fin():
            mean = sums[...] / jnp.maximum(counts[...][:, 0:1], 1.0)
            o_ref[...] = jnp.dot(mean, wl_ref[...],
                                 preferred_element_type=jnp.float32) + bl_ref[...]

    return pl.pallas_call(
        body,
        grid=(nblk,),
        in_specs=[
            pl.BlockSpec((1, blk, feat), lambda i: (0, i, 0)),
            pl.BlockSpec((1, blk, feat), lambda i: (1, i, 0)),
            pl.BlockSpec((blk, feat), lambda i: (i, 0)),
            pl.BlockSpec((feat, feat), lambda i: (0, 0)),
            pl.BlockSpec((1, 1, blk), lambda i: (i, 0, 0)),
            pl.BlockSpec((feat, feat), lambda i: (0, 0)),
            pl.BlockSpec((1, feat), lambda i: (0, 0)),
        ],
        out_specs=pl.BlockSpec((num_graphs, feat), lambda i: (0, 0)),
        out_shape=jax.ShapeDtypeStruct((num_graphs, feat), jnp.float32),
        scratch_shapes=[
            pltpu.VMEM((num_graphs, feat), jnp.float32),
            pltpu.VMEM((num_graphs, 8), jnp.float32),
        ],
    )(agg2, agg2, r, w_rel_t, batch_r, w_lin_t_pad, b_lin_pad.reshape(1, feat))


def kernel(x, edge_index, edge_attr, batch, W1_rel, b1_rel, W1_root,
           W2_rel, b2_rel, W2_root, W3_rel, b3_rel, W3_root, W_lin, b_lin):
    del edge_attr
    n, feat = x.shape
    num_graphs = 512
    n_classes = W_lin.shape[0]

    src_flat = edge_index[0]
    dst_flat = edge_index[1]
    zeros_rows = jnp.zeros((1000, feat), jnp.float32)
    batch_r = batch.reshape(n // 1000, 1, 1000)

    w_lin_t_pad = jnp.zeros((feat, feat), jnp.float32).at[:, :n_classes].set(W_lin.T)
    b_lin_pad = jnp.zeros((feat,), jnp.float32).at[:n_classes].set(b_lin)

    r = _tc_root(x, W1_root.T, b1_rel)
    agg2 = _sc_segsum(x, src_flat, dst_flat, zeros_rows, n, feat)
    h, r = _tc_combine_root(agg2, r, W1_rel.T, W2_root.T, b2_rel)
    agg2 = _sc_segsum(h, src_flat, dst_flat, zeros_rows, n, feat)
    h, r = _tc_combine_root(agg2, r, W2_rel.T, W3_root.T, b3_rel)
    agg2 = _sc_segsum(h, src_flat, dst_flat, zeros_rows, n, feat)
    pooled = _tc_combine_pool_linear(
        agg2, r, W3_rel.T, batch_r, w_lin_t_pad, b_lin_pad, num_graphs)

    return pooled[:, :n_classes]


# speedup vs baseline: 1.0056x; 1.0056x over previous
import functools

import jax
import jax.numpy as jnp
from jax import lax
from jax.experimental import pallas as pl
from jax.experimental.pallas import tpu as pltpu
from jax.experimental.pallas import tpu_sc as plsc

_NC = 2
_NS = 16
_CHUNK = 80
_GRP = 2


def _sc_segsum(xf, src_flat, dst_flat, zeros_rows, n_nodes, feat):
    num_edges = src_flat.shape[0]
    e_tile = num_edges // (_NC * _NS)
    n_chunks = e_tile // _CHUNK
    n_groups = n_chunks // _GRP
    n_tail = n_chunks - n_groups * _GRP
    mesh = plsc.VectorSubcoreMesh(
        core_axis_name="c", subcore_axis_name="s",
        num_cores=_NC, num_subcores=_NS)

    @functools.partial(
        pl.kernel,
        out_type=jax.ShapeDtypeStruct((_NC, n_nodes, feat), jnp.float32),
        mesh=mesh,
        scratch_types=[
            pltpu.VMEM_SHARED((n_nodes, feat), jnp.float32),
            pltpu.VMEM((4 * _GRP, _CHUNK), jnp.int32),
            pltpu.VMEM((4 * _GRP, _CHUNK), jnp.int32),
            pltpu.VMEM((2 * _GRP, _CHUNK, feat), jnp.float32),
            pltpu.SemaphoreType.DMA, pltpu.SemaphoreType.DMA,
            pltpu.SemaphoreType.DMA, pltpu.SemaphoreType.DMA,
            pltpu.SemaphoreType.DMA, pltpu.SemaphoreType.DMA,
        ],
    )
    def k(x_hbm, src_hbm, dst_hbm, z_hbm, out_hbm, agg,
          src_v, dst_v, bufs, isem0, isem1, gsem0, gsem1, ssem0, ssem1):
        c = lax.axis_index("c")
        s = lax.axis_index("s")
        base = (c * _NS + s) * e_tile

        @pl.when(s < n_nodes // 1000)
        def _zero():
            zbase = pl.multiple_of(s * 1000, 8)
            pltpu.sync_copy(z_hbm, agg.at[pl.ds(zbase, 1000)])

        plsc.subcore_barrier()

        def fetch_idx(g, isem):
            r = (g % 4) * _GRP
            for b in range(_GRP):
                off = base + (g * _GRP + b) * _CHUNK
                pltpu.async_copy(src_hbm.at[pl.ds(off, _CHUNK)],
                                 src_v.at[r + b], isem)
                pltpu.async_copy(dst_hbm.at[pl.ds(off, _CHUNK)],
                                 dst_v.at[r + b], isem)

        def wait_idx(g, isem):
            r = (g % 4) * _GRP
            for b in range(_GRP):
                pltpu.make_async_copy(src_hbm.at[pl.ds(base, _CHUNK)],
                                      src_v.at[r + b], isem).wait()
                pltpu.make_async_copy(dst_hbm.at[pl.ds(base, _CHUNK)],
                                      dst_v.at[r + b], isem).wait()

        def fire_rows(g, gsem):
            ri = (g % 4) * _GRP
            rb = (g % 2) * _GRP
            for b in range(_GRP):
                pltpu.async_copy(x_hbm.at[src_v.at[ri + b]],
                                 bufs.at[rb + b], gsem)

        def wait_rows(g, gsem):
            ri = (g % 4) * _GRP
            rb = (g % 2) * _GRP
            for b in range(_GRP):
                pltpu.make_async_copy(x_hbm.at[src_v.at[ri + b]],
                                      bufs.at[rb + b], gsem).wait()

        def fire_scatters(g, ssem):
            ri = (g % 4) * _GRP
            rb = (g % 2) * _GRP
            for b in range(_GRP):
                pltpu.async_copy(bufs.at[rb + b],
                                 agg.at[dst_v.at[ri + b]], ssem, add=True)

        def wait_scatters(g, ssem):
            ri = (g % 4) * _GRP
            rb = (g % 2) * _GRP
            for b in range(_GRP):
                pltpu.make_async_copy(bufs.at[rb + b],
                                      agg.at[dst_v.at[ri + b]], ssem).wait()

        fetch_idx(0, isem0)
        fetch_idx(1, isem1)
        wait_idx(0, isem0)
        fire_rows(0, gsem0)

        def group_body(g, carry):
            def run(gsem, ssem, gsem_o, ssem_o, isem_n, isem_n2):
                wait_rows(g, gsem)
                fire_scatters(g, ssem)

                @pl.when(g >= 1)
                def _free_other():
                    wait_scatters(g - 1, ssem_o)

                @pl.when(g + 1 < n_groups)
                def _next_rows():
                    wait_idx(g + 1, isem_n)
                    fire_rows(g + 1, gsem_o)

                @pl.when(g + 2 < n_groups)
                def _next_idx():
                    fetch_idx(g + 2, isem_n2)

            @pl.when(g % 2 == 0)
            def _even():
                run(gsem0, ssem0, gsem1, ssem1, isem1, isem0)

            @pl.when(g % 2 == 1)
            def _odd():
                run(gsem1, ssem1, gsem0, ssem0, isem0, isem1)

            return carry

        lax.fori_loop(0, n_groups, group_body, 0)
        last = n_groups - 1
        wait_scatters(last, ssem1 if last % 2 == 1 else ssem0)
        for t in range(n_tail):
            off = base + (n_groups * _GRP + t) * _CHUNK
            pltpu.sync_copy(src_hbm.at[pl.ds(off, _CHUNK)], src_v.at[0])
            pltpu.sync_copy(dst_hbm.at[pl.ds(off, _CHUNK)], dst_v.at[0])
            pltpu.sync_copy(x_hbm.at[src_v.at[0]], bufs.at[0])
            pltpu.sync_copy(bufs.at[0], agg.at[dst_v.at[0]], add=True)
        plsc.subcore_barrier()

        @pl.when(s < n_nodes // 1000)
        def _writeback():
            wbase = pl.multiple_of(s * 1000, 8)
            pltpu.sync_copy(agg.at[pl.ds(wbase, 1000)],
                            out_hbm.at[c, pl.ds(wbase, 1000)])

    return k(xf, src_flat, dst_flat, zeros_rows)


def _tc_root(h, w_root_t, bias, blk=1000):
    n, feat = h.shape

    def body(h_ref, w_ref, b_ref, o_ref):
        o_ref[...] = jnp.dot(h_ref[...], w_ref[...],
                             preferred_element_type=jnp.float32) + b_ref[...]

    return pl.pallas_call(
        body,
        grid=(n // blk,),
        in_specs=[
            pl.BlockSpec((blk, feat), lambda i: (i, 0)),
            pl.BlockSpec((feat, feat), lambda i: (0, 0)),
            pl.BlockSpec((1, feat), lambda i: (0, 0)),
        ],
        out_specs=pl.BlockSpec((blk, feat), lambda i: (i, 0)),
        out_shape=jax.ShapeDtypeStruct((n, feat), jnp.float32),
    )(h, w_root_t, bias.reshape(1, feat))


def _tc_combine(agg2, r, w_rel_t, blk=1000):
    n, feat = r.shape

    def body(a0_ref, a1_ref, r_ref, w_ref, o_ref):
        acc = jnp.dot(a0_ref[0] + a1_ref[0], w_ref[...],
                      preferred_element_type=jnp.float32)
        o_ref[...] = jnp.maximum(acc + r_ref[...], 0.0)

    return pl.pallas_call(
        body,
        grid=(n // blk,),
        in_specs=[
            pl.BlockSpec((1, blk, feat), lambda i: (0, i, 0)),
            pl.BlockSpec((1, blk, feat), lambda i: (1, i, 0)),
            pl.BlockSpec((blk, feat), lambda i: (i, 0)),
            pl.BlockSpec((feat, feat), lambda i: (0, 0)),
        ],
        out_specs=pl.BlockSpec((blk, feat), lambda i: (i, 0)),
        out_shape=jax.ShapeDtypeStruct((n, feat), jnp.float32),
    )(agg2, agg2, r, w_rel_t)


def _tc_combine_pool_linear(agg2, r, w_rel_t, batch_r, w_lin_t_pad, b_lin_pad,
                            num_graphs, blk=1000):
    n, feat = r.shape
    nblk = n // blk

    def body(a0_ref, a1_ref, r_ref, w_ref, bid_ref, wl_ref, bl_ref,
             o_ref, sums, counts):
        i = pl.program_id(0)

        @pl.when(i == 0)
        def _init():
            sums[...] = jnp.zeros_like(sums)
            counts[...] = jnp.zeros_like(counts)

        h_blk = jnp.dot(a0_ref[0] + a1_ref[0], w_ref[...],
                        preferred_element_type=jnp.float32) + r_ref[...]

        bid = bid_ref[0]
        gids = lax.broadcasted_iota(jnp.int32, (num_graphs, blk), 0)
        onehot_t = (gids == bid).astype(jnp.float32)
        sums[...] += lax.dot_general(
            onehot_t, h_blk, (((1,), (0,)), ((), ())),
            preferred_element_type=jnp.float32)
        counts[...] += lax.dot_general(
            onehot_t, jnp.ones((blk, 8), jnp.float32),
            (((1,), (0,)), ((), ())), preferred_element_type=jnp.float32)

        @pl.when(i == nblk - 1)
        def _fin():
            mean = sums[...] / jnp.maximum(counts[...][:, 0:1], 1.0)
            o_ref[...] = jnp.dot(mean, wl_ref[...],
                                 preferred_element_type=jnp.float32) + bl_ref[...]

    return pl.pallas_call(
        body,
        grid=(nblk,),
        in_specs=[
            pl.BlockSpec((1, blk, feat), lambda i: (0, i, 0)),
            pl.BlockSpec((1, blk, feat), lambda i: (1, i, 0)),
            pl.BlockSpec((blk, feat), lambda i: (i, 0)),
            pl.BlockSpec((feat, feat), lambda i: (0, 0)),
            pl.BlockSpec((1, 1, blk), lambda i: (i, 0, 0)),
            pl.BlockSpec((feat, feat), lambda i: (0, 0)),
            pl.BlockSpec((1, feat), lambda i: (0, 0)),
        ],
        out_specs=pl.BlockSpec((num_graphs, feat), lambda i: (0, 0)),
        out_shape=jax.ShapeDtypeStruct((num_graphs, feat), jnp.float32),
        scratch_shapes=[
            pltpu.VMEM((num_graphs, feat), jnp.float32),
            pltpu.VMEM((num_graphs, 8), jnp.float32),
        ],
    )(agg2, agg2, r, w_rel_t, batch_r, w_lin_t_pad, b_lin_pad.reshape(1, feat))


def kernel(x, edge_index, edge_attr, batch, W1_rel, b1_rel, W1_root,
           W2_rel, b2_rel, W2_root, W3_rel, b3_rel, W3_root, W_lin, b_lin):
    del edge_attr
    n, feat = x.shape
    num_graphs = 512
    n_classes = W_lin.shape[0]

    src_flat = edge_index[0]
    dst_flat = edge_index[1]
    zeros_rows = jnp.zeros((1000, feat), jnp.float32)
    batch_r = batch.reshape(n // 1000, 1, 1000)

    w_lin_t_pad = jnp.zeros((feat, feat), jnp.float32).at[:, :n_classes].set(W_lin.T)
    b_lin_pad = jnp.zeros((feat,), jnp.float32).at[:n_classes].set(b_lin)

    h = x
    for w_rel, b_rel, w_root, layer in (
            (W1_rel, b1_rel, W1_root, 1),
            (W2_rel, b2_rel, W2_root, 2),
            (W3_rel, b3_rel, W3_root, 3)):
        r = _tc_root(h, w_root.T, b_rel)
        agg2 = _sc_segsum(h, src_flat, dst_flat, zeros_rows, n, feat)
        if layer < 3:
            h = _tc_combine(agg2, r, w_rel.T)
        else:
            pooled = _tc_combine_pool_linear(
                agg2, r, w_rel.T, batch_r, w_lin_t_pad, b_lin_pad, num_graphs)

    return pooled[:, :n_classes]


# speedup vs baseline: 1.0095x; 1.0039x over previous
import functools

import jax
import jax.numpy as jnp
from jax import lax
from jax.experimental import pallas as pl
from jax.experimental.pallas import tpu as pltpu
from jax.experimental.pallas import tpu_sc as plsc

_NC = 2
_NS = 16
_CHUNK = 80
_GRP = 2


def _sc_segsum(xf, src_flat, dst_flat, zeros_rows, n_nodes, feat):
    num_edges = src_flat.shape[0]
    e_tile = num_edges // (_NC * _NS)
    n_chunks = e_tile // _CHUNK
    n_groups = n_chunks // _GRP
    n_tail = n_chunks - n_groups * _GRP
    mesh = plsc.VectorSubcoreMesh(
        core_axis_name="c", subcore_axis_name="s",
        num_cores=_NC, num_subcores=_NS)

    @functools.partial(
        pl.kernel,
        out_type=jax.ShapeDtypeStruct((_NC, n_nodes, feat), jnp.float32),
        mesh=mesh,
        scratch_types=[
            pltpu.VMEM_SHARED((n_nodes, feat), jnp.float32),
            pltpu.VMEM((4 * _GRP, _CHUNK), jnp.int32),
            pltpu.VMEM((4 * _GRP, _CHUNK), jnp.int32),
            pltpu.VMEM((2 * _GRP, _CHUNK, feat), jnp.float32),
            pltpu.SemaphoreType.DMA, pltpu.SemaphoreType.DMA,
            pltpu.SemaphoreType.DMA, pltpu.SemaphoreType.DMA,
            pltpu.SemaphoreType.DMA, pltpu.SemaphoreType.DMA,
        ],
    )
    def k(x_hbm, src_hbm, dst_hbm, z_hbm, out_hbm, agg,
          src_v, dst_v, bufs, isem0, isem1, gsem0, gsem1, ssem0, ssem1):
        c = lax.axis_index("c")
        s = lax.axis_index("s")
        base = (c * _NS + s) * e_tile

        def fetch_idx(g, isem):
            r = (g % 4) * _GRP
            for b in range(_GRP):
                off = base + (g * _GRP + b) * _CHUNK
                pltpu.async_copy(src_hbm.at[pl.ds(off, _CHUNK)],
                                 src_v.at[r + b], isem)
                pltpu.async_copy(dst_hbm.at[pl.ds(off, _CHUNK)],
                                 dst_v.at[r + b], isem)

        def wait_idx(g, isem):
            r = (g % 4) * _GRP
            for b in range(_GRP):
                pltpu.make_async_copy(src_hbm.at[pl.ds(base, _CHUNK)],
                                      src_v.at[r + b], isem).wait()
                pltpu.make_async_copy(dst_hbm.at[pl.ds(base, _CHUNK)],
                                      dst_v.at[r + b], isem).wait()

        def fire_rows(g, gsem):
            ri = (g % 4) * _GRP
            rb = (g % 2) * _GRP
            for b in range(_GRP):
                pltpu.async_copy(x_hbm.at[src_v.at[ri + b]],
                                 bufs.at[rb + b], gsem)

        def wait_rows(g, gsem):
            ri = (g % 4) * _GRP
            rb = (g % 2) * _GRP
            for b in range(_GRP):
                pltpu.make_async_copy(x_hbm.at[src_v.at[ri + b]],
                                      bufs.at[rb + b], gsem).wait()

        def fire_scatters(g, ssem):
            ri = (g % 4) * _GRP
            rb = (g % 2) * _GRP
            for b in range(_GRP):
                pltpu.async_copy(bufs.at[rb + b],
                                 agg.at[dst_v.at[ri + b]], ssem, add=True)

        def wait_scatters(g, ssem):
            ri = (g % 4) * _GRP
            rb = (g % 2) * _GRP
            for b in range(_GRP):
                pltpu.make_async_copy(bufs.at[rb + b],
                                      agg.at[dst_v.at[ri + b]], ssem).wait()

        fetch_idx(0, isem0)
        fetch_idx(1, isem1)

        @pl.when(s < n_nodes // 1000)
        def _zero():
            zbase = pl.multiple_of(s * 1000, 8)
            pltpu.sync_copy(z_hbm, agg.at[pl.ds(zbase, 1000)])

        wait_idx(0, isem0)
        fire_rows(0, gsem0)
        plsc.subcore_barrier()

        def group_body(g, carry):
            def run(gsem, ssem, gsem_o, ssem_o, isem_n, isem_n2):
                wait_rows(g, gsem)
                fire_scatters(g, ssem)

                @pl.when(g >= 1)
                def _free_other():
                    wait_scatters(g - 1, ssem_o)

                @pl.when(g + 1 < n_groups)
                def _next_rows():
                    wait_idx(g + 1, isem_n)
                    fire_rows(g + 1, gsem_o)

                @pl.when(g + 2 < n_groups)
                def _next_idx():
                    fetch_idx(g + 2, isem_n2)

            @pl.when(g % 2 == 0)
            def _even():
                run(gsem0, ssem0, gsem1, ssem1, isem1, isem0)

            @pl.when(g % 2 == 1)
            def _odd():
                run(gsem1, ssem1, gsem0, ssem0, isem0, isem1)

            return carry

        lax.fori_loop(0, n_groups, group_body, 0)
        last = n_groups - 1
        wait_scatters(last, ssem1 if last % 2 == 1 else ssem0)
        for t in range(n_tail):
            off = base + (n_groups * _GRP + t) * _CHUNK
            pltpu.sync_copy(src_hbm.at[pl.ds(off, _CHUNK)], src_v.at[0])
            pltpu.sync_copy(dst_hbm.at[pl.ds(off, _CHUNK)], dst_v.at[0])
            pltpu.sync_copy(x_hbm.at[src_v.at[0]], bufs.at[0])
            pltpu.sync_copy(bufs.at[0], agg.at[dst_v.at[0]], add=True)
        plsc.subcore_barrier()

        @pl.when(s < n_nodes // 1000)
        def _writeback():
            wbase = pl.multiple_of(s * 1000, 8)
            pltpu.sync_copy(agg.at[pl.ds(wbase, 1000)],
                            out_hbm.at[c, pl.ds(wbase, 1000)])

    return k(xf, src_flat, dst_flat, zeros_rows)


def _tc_root(h, w_root_t, bias, blk=1000):
    n, feat = h.shape

    def body(h_ref, w_ref, b_ref, o_ref):
        o_ref[...] = jnp.dot(h_ref[...], w_ref[...],
                             preferred_element_type=jnp.float32) + b_ref[...]

    return pl.pallas_call(
        body,
        grid=(n // blk,),
        in_specs=[
            pl.BlockSpec((blk, feat), lambda i: (i, 0)),
            pl.BlockSpec((feat, feat), lambda i: (0, 0)),
            pl.BlockSpec((1, feat), lambda i: (0, 0)),
        ],
        out_specs=pl.BlockSpec((blk, feat), lambda i: (i, 0)),
        out_shape=jax.ShapeDtypeStruct((n, feat), jnp.float32),
    )(h, w_root_t, bias.reshape(1, feat))


def _tc_combine(agg2, r, w_rel_t, blk=1000):
    n, feat = r.shape

    def body(a0_ref, a1_ref, r_ref, w_ref, o_ref):
        acc = jnp.dot(a0_ref[0] + a1_ref[0], w_ref[...],
                      preferred_element_type=jnp.float32)
        o_ref[...] = jnp.maximum(acc + r_ref[...], 0.0)

    return pl.pallas_call(
        body,
        grid=(n // blk,),
        in_specs=[
            pl.BlockSpec((1, blk, feat), lambda i: (0, i, 0)),
            pl.BlockSpec((1, blk, feat), lambda i: (1, i, 0)),
            pl.BlockSpec((blk, feat), lambda i: (i, 0)),
            pl.BlockSpec((feat, feat), lambda i: (0, 0)),
        ],
        out_specs=pl.BlockSpec((blk, feat), lambda i: (i, 0)),
        out_shape=jax.ShapeDtypeStruct((n, feat), jnp.float32),
    )(agg2, agg2, r, w_rel_t)


def _tc_combine_pool_linear(agg2, r, w_rel_t, batch_r, w_lin_t_pad, b_lin_pad,
                            num_graphs, blk=1000):
    n, feat = r.shape
    nblk = n // blk

    def body(a0_ref, a1_ref, r_ref, w_ref, bid_ref, wl_ref, bl_ref,
             o_ref, sums, counts):
        i = pl.program_id(0)

        @pl.when(i == 0)
        def _init():
            sums[...] = jnp.zeros_like(sums)
            counts[...] = jnp.zeros_like(counts)

        h_blk = jnp.dot(a0_ref[0] + a1_ref[0], w_ref[...],
                        preferred_element_type=jnp.float32) + r_ref[...]

        bid = bid_ref[0]
        gids = lax.broadcasted_iota(jnp.int32, (num_graphs, blk), 0)
        onehot_t = (gids == bid).astype(jnp.float32)
        sums[...] += lax.dot_general(
            onehot_t, h_blk, (((1,), (0,)), ((), ())),
            preferred_element_type=jnp.float32)
        counts[...] += lax.dot_general(
            onehot_t, jnp.ones((blk, 8), jnp.float32),
            (((1,), (0,)), ((), ())), preferred_element_type=jnp.float32)

        @pl.when(i == nblk - 1)
        def _fin():
            mean = sums[...] / jnp.maximum(counts[...][:, 0:1], 1.0)
            o_ref[...] = jnp.dot(mean, wl_ref[...],
                                 preferred_element_type=jnp.float32) + bl_ref[...]

    return pl.pallas_call(
        body,
        grid=(nblk,),
        in_specs=[
            pl.BlockSpec((1, blk, feat), lambda i: (0, i, 0)),
            pl.BlockSpec((1, blk, feat), lambda i: (1, i, 0)),
            pl.BlockSpec((blk, feat), lambda i: (i, 0)),
            pl.BlockSpec((feat, feat), lambda i: (0, 0)),
            pl.BlockSpec((1, 1, blk), lambda i: (i, 0, 0)),
            pl.BlockSpec((feat, feat), lambda i: (0, 0)),
            pl.BlockSpec((1, feat), lambda i: (0, 0)),
        ],
        out_specs=pl.BlockSpec((num_graphs, feat), lambda i: (0, 0)),
        out_shape=jax.ShapeDtypeStruct((num_graphs, feat), jnp.float32),
        scratch_shapes=[
            pltpu.VMEM((num_graphs, feat), jnp.float32),
            pltpu.VMEM((num_graphs, 8), jnp.float32),
        ],
    )(agg2, agg2, r, w_rel_t, batch_r, w_lin_t_pad, b_lin_pad.reshape(1, feat))


def kernel(x, edge_index, edge_attr, batch, W1_rel, b1_rel, W1_root,
           W2_rel, b2_rel, W2_root, W3_rel, b3_rel, W3_root, W_lin, b_lin):
    del edge_attr
    n, feat = x.shape
    num_graphs = 512
    n_classes = W_lin.shape[0]

    src_flat = edge_index[0]
    dst_flat = edge_index[1]
    zeros_rows = jnp.zeros((1000, feat), jnp.float32)
    batch_r = batch.reshape(n // 1000, 1, 1000)

    w_lin_t_pad = jnp.zeros((feat, feat), jnp.float32).at[:, :n_classes].set(W_lin.T)
    b_lin_pad = jnp.zeros((feat,), jnp.float32).at[:n_classes].set(b_lin)

    h = x
    for w_rel, b_rel, w_root, layer in (
            (W1_rel, b1_rel, W1_root, 1),
            (W2_rel, b2_rel, W2_root, 2),
            (W3_rel, b3_rel, W3_root, 3)):
        r = _tc_root(h, w_root.T, b_rel)
        agg2 = _sc_segsum(h, src_flat, dst_flat, zeros_rows, n, feat)
        if layer < 3:
            h = _tc_combine(agg2, r, w_rel.T)
        else:
            pooled = _tc_combine_pool_linear(
                agg2, r, w_rel.T, batch_r, w_lin_t_pad, b_lin_pad, num_graphs)

    return pooled[:, :n_classes]
